# baseline (speedup 1.0000x reference)
import functools

import jax
import jax.numpy as jnp
from jax import lax
from jax.experimental import pallas as pl
from jax.experimental.pallas import tpu as pltpu
from jax.experimental.pallas import tpu_sc as plsc

N_NODES = 10000
N_EDGES = 320000
D_IN = 128
D_HID = 16

NC = 2
NS = 16
NW = NC * NS
CH = 125
NCH = 80
E_PER_W = NCH * CH
NBUF = 10
NOUT = NCH // NBUF
N_PAD = 10240
ZR = N_PAD // NS
PK = N_PAD // 8
PKV = N_NODES // 8


def _project_kernel(x_ref, w_ref, o_ref):
    o_ref[pl.ds(0, PKV), :] = jnp.dot(
        x_ref[...], w_ref[...], preferred_element_type=jnp.float32)
    o_ref[pl.ds(PKV, PK - PKV), :] = jnp.zeros((PK - PKV, 128), jnp.float32)


def _mlp_kernel(y_ref, p0_ref, p1_ref, w2_ref, b1_ref, b2_ref, s_ref, o_ref):
    s = s_ref[0, 0]
    h = s * y_ref[...] + (p0_ref[...] + p1_ref[...]) + b1_ref[...]
    h = jnp.maximum(h[:PKV, :], 0.0)
    h = jnp.dot(h, w2_ref[...], preferred_element_type=jnp.float32) + b2_ref[...]
    o_ref[...] = jnp.maximum(h, 0.0)


@functools.partial(
    pl.kernel,
    out_type=(jax.ShapeDtypeStruct((N_PAD, D_HID), jnp.float32),
              jax.ShapeDtypeStruct((N_PAD, D_HID), jnp.float32)),
    mesh=plsc.VectorSubcoreMesh(core_axis_name="c", subcore_axis_name="s"),
    scratch_types=[
        pltpu.VMEM((NCH, CH), jnp.int32),
        pltpu.VMEM((NCH, CH), jnp.int32),
        pltpu.VMEM((NBUF, CH, D_HID), jnp.float32),
        pltpu.VMEM((ZR, D_HID), jnp.float32),
        pltpu.VMEM_SHARED((N_PAD, D_HID), jnp.float32),
        pltpu.SemaphoreType.DMA((NBUF,)),
    ],
    compiler_params=pltpu.CompilerParams(use_tc_tiling_on_sc=False),
)
def _sc_aggregate(y_hbm, edges_hbm, zeros_hbm, p0_hbm, p1_hbm,
                  src_v, dst_v, rows_v, stage_v, acc, sems):
    cid = lax.axis_index("c")
    sid = lax.axis_index("s")
    wid = cid * NS + sid

    pltpu.async_copy(edges_hbm.at[0, wid], src_v, sems.at[0])
    pltpu.async_copy(edges_hbm.at[1, wid], dst_v, sems.at[1])

    pltpu.sync_copy(zeros_hbm, stage_v)
    pltpu.sync_copy(stage_v, acc.at[pl.ds(sid * ZR, ZR)])

    pltpu.make_async_copy(edges_hbm.at[0, wid], src_v, sems.at[0]).wait()
    pltpu.make_async_copy(edges_hbm.at[1, wid], dst_v, sems.at[1]).wait()

    for b in range(NBUF):
        pltpu.async_copy(y_hbm.at[src_v.at[b]], rows_v.at[b], sems.at[b])
    plsc.subcore_barrier()

    def body(g, carry):
        for b in range(NBUF):
            c = g * NBUF + b
            pltpu.make_async_copy(
                y_hbm.at[src_v.at[c]], rows_v.at[b], sems.at[b]).wait()
            pltpu.sync_copy(rows_v.at[b], acc.at[dst_v.at[c]], add=True)

            @pl.when(g < NOUT - 1)
            def _():
                pltpu.async_copy(
                    y_hbm.at[src_v.at[c + NBUF]], rows_v.at[b], sems.at[b])

        return carry

    lax.fori_loop(0, NOUT, body, 0)

    plsc.subcore_barrier()
    pltpu.sync_copy(acc.at[pl.ds(sid * ZR, ZR)], stage_v)

    @pl.when(cid == 0)
    def _():
        pltpu.sync_copy(stage_v, p0_hbm.at[pl.ds(sid * ZR, ZR)])

    @pl.when(cid == 1)
    def _():
        pltpu.sync_copy(stage_v, p1_hbm.at[pl.ds(sid * ZR, ZR)])


def kernel(x, edge_index, W1, b1, W2, b2, eps):
    f32 = jnp.float32
    eye8 = jnp.eye(8, dtype=f32)

    y8 = pl.pallas_call(
        _project_kernel,
        out_shape=jax.ShapeDtypeStruct((PK, 128), f32),
    )(x.reshape(PKV, 8 * D_IN), jnp.kron(eye8, W1))

    e4 = edge_index.reshape(2, NW, NCH, CH)
    p0, p1 = _sc_aggregate(y8.reshape(N_PAD, D_HID), e4,
                           jnp.zeros((ZR, D_HID), f32))

    scale = (1.0 + eps).reshape(1, 1)
    out8 = pl.pallas_call(
        _mlp_kernel,
        out_shape=jax.ShapeDtypeStruct((PKV, 128), f32),
    )(y8, p0.reshape(PK, 128), p1.reshape(PK, 128),
      jnp.kron(eye8, W2), jnp.tile(b1, 8).reshape(1, 128),
      jnp.tile(b2, 8).reshape(1, 128), scale)
    return out8.reshape(N_NODES, D_HID)


# speedup vs baseline: 1.0741x; 1.0741x over previous
import functools

import jax
import jax.numpy as jnp
from jax import lax
from jax.experimental import pallas as pl
from jax.experimental.pallas import tpu as pltpu
from jax.experimental.pallas import tpu_sc as plsc

N_NODES = 10000
N_EDGES = 320000
D_IN = 128
D_HID = 16

NC = 2
NS = 16
NW = NC * NS
CH = 125
NCH = 80
E_PER_W = NCH * CH
NBUF = 8
NOUT = NCH // NBUF
N_PAD = 10240
ZR = N_PAD // NS
PK = N_PAD // 8
PKV = N_NODES // 8


def _project_kernel(x_ref, w_ref, o_ref):
    for j in range(8):
        o_ref[pl.ds(0, PKV), 16 * j:16 * (j + 1)] = jnp.dot(
            x_ref[j::8, :], w_ref[...], preferred_element_type=jnp.float32)
    o_ref[pl.ds(PKV, PK - PKV), :] = jnp.zeros((PK - PKV, 128), jnp.float32)


def _mlp_kernel(y_ref, p0_ref, p1_ref, w2_ref, b1_ref, b2_ref, s_ref, o_ref):
    s = s_ref[0, 0]
    h = s * y_ref[...] + (p0_ref[...] + p1_ref[...]) + b1_ref[...]
    h = jnp.maximum(h[:PKV, :], 0.0)
    h = jnp.dot(h, w2_ref[...], preferred_element_type=jnp.float32) + b2_ref[...]
    o_ref[...] = jnp.maximum(h, 0.0)


@functools.partial(
    pl.kernel,
    out_type=(jax.ShapeDtypeStruct((N_PAD, D_HID), jnp.float32),
              jax.ShapeDtypeStruct((N_PAD, D_HID), jnp.float32)),
    mesh=plsc.VectorSubcoreMesh(core_axis_name="c", subcore_axis_name="s"),
    scratch_types=[
        pltpu.VMEM((NCH, CH), jnp.int32),
        pltpu.VMEM((NCH, CH), jnp.int32),
        pltpu.VMEM((NBUF, CH, D_HID), jnp.float32),
        pltpu.VMEM((ZR, D_HID), jnp.float32),
        pltpu.VMEM_SHARED((N_PAD, D_HID), jnp.float32),
        pltpu.SemaphoreType.DMA((NBUF,)),
    ],
    compiler_params=pltpu.CompilerParams(use_tc_tiling_on_sc=False),
)
def _sc_aggregate(y_hbm, edges_hbm, zeros_hbm, p0_hbm, p1_hbm,
                  src_v, dst_v, rows_v, stage_v, acc, sems):
    cid = lax.axis_index("c")
    sid = lax.axis_index("s")
    wid = cid * NS + sid

    pltpu.async_copy(edges_hbm.at[0, wid], src_v, sems.at[0])
    pltpu.async_copy(edges_hbm.at[1, wid], dst_v, sems.at[1])

    pltpu.sync_copy(zeros_hbm, stage_v)
    pltpu.sync_copy(stage_v, acc.at[pl.ds(sid * ZR, ZR)])

    pltpu.make_async_copy(edges_hbm.at[0, wid], src_v, sems.at[0]).wait()
    pltpu.make_async_copy(edges_hbm.at[1, wid], dst_v, sems.at[1]).wait()

    for b in range(NBUF):
        pltpu.async_copy(y_hbm.at[src_v.at[b]], rows_v.at[b], sems.at[b])
    plsc.subcore_barrier()

    def body(g, carry):
        for b in range(NBUF):
            c = g * NBUF + b
            pltpu.make_async_copy(
                y_hbm.at[src_v.at[c]], rows_v.at[b], sems.at[b]).wait()
            pltpu.sync_copy(rows_v.at[b], acc.at[dst_v.at[c]], add=True)

            @pl.when(g < NOUT - 1)
            def _():
                pltpu.async_copy(
                    y_hbm.at[src_v.at[c + NBUF]], rows_v.at[b], sems.at[b])

        return carry

    lax.fori_loop(0, NOUT, body, 0)

    plsc.subcore_barrier()
    pltpu.sync_copy(acc.at[pl.ds(sid * ZR, ZR)], stage_v)

    @pl.when(cid == 0)
    def _():
        pltpu.sync_copy(stage_v, p0_hbm.at[pl.ds(sid * ZR, ZR)])

    @pl.when(cid == 1)
    def _():
        pltpu.sync_copy(stage_v, p1_hbm.at[pl.ds(sid * ZR, ZR)])


def kernel(x, edge_index, W1, b1, W2, b2, eps):
    f32 = jnp.float32
    eye8 = jnp.eye(8, dtype=f32)

    y8 = pl.pallas_call(
        _project_kernel,
        out_shape=jax.ShapeDtypeStruct((PK, 128), f32),
    )(x, W1)

    e4 = edge_index.reshape(2, NW, NCH, CH)
    p0, p1 = _sc_aggregate(y8.reshape(N_PAD, D_HID), e4,
                           jnp.zeros((ZR, D_HID), f32))

    scale = (1.0 + eps).reshape(1, 1)
    out8 = pl.pallas_call(
        _mlp_kernel,
        out_shape=jax.ShapeDtypeStruct((PKV, 128), f32),
    )(y8, p0.reshape(PK, 128), p1.reshape(PK, 128),
      jnp.kron(eye8, W2), jnp.tile(b1, 8).reshape(1, 128),
      jnp.tile(b2, 8).reshape(1, 128), scale)
    return out8.reshape(N_NODES, D_HID)
